# Initial kernel scaffold; baseline (speedup 1.0000x reference)
import jax
import jax.numpy as jnp
from jax.experimental import pallas as pl


def kernel(x, edge_index, W, b):
    raise NotImplementedError("write your pallas kernel here")


# speedup vs baseline: 28.4696x; 28.4696x over previous
import functools

import jax
import jax.numpy as jnp
from jax import lax
from jax.experimental import pallas as pl
from jax.experimental.pallas import tpu as pltpu
from jax.experimental.pallas import tpu_sc as plsc

NC, NS = 2, 16
NW = NC * NS
K = 80


def _deg_kernel_body(N_pad, n_chunks, dst_hbm, out_hbm, dst_v, ones_v, zero_v,
                     deg_sh, sem):
    c = lax.axis_index("c")
    sid = lax.axis_index("s")
    wid = c * NS + sid

    if True:
        pltpu.async_copy(dst_hbm.at[wid], dst_v, sem).wait()
        zchunk = N_pad // NS

        def zbody(i, carry):
            zero_v[pl.ds(i * 16, 16)] = jnp.zeros((16,), jnp.float32)
            return carry

        lax.fori_loop(0, zchunk // 16, zbody, 0, unroll=8)
        pltpu.sync_copy(zero_v, deg_sh.at[pl.ds(sid * zchunk, zchunk)])
        for i in range(K // 16):
            ones_v[pl.ds(i * 16, 16)] = jnp.ones((16,), jnp.float32)
        plsc.subcore_barrier()

        def body(j, carry):
            pltpu.sync_copy(ones_v, deg_sh.at[dst_v.at[j]], add=True)
            return carry

        lax.fori_loop(0, n_chunks, body, 0, unroll=4)
        plsc.subcore_barrier()
        pltpu.sync_copy(deg_sh.at[pl.ds(sid * zchunk, zchunk)],
                        out_hbm.at[c, pl.ds(sid * zchunk, zchunk)])


def _edge_kernel_body(N, N_pad, n_chunks, hs_hbm, src_hbm, dst_hbm, out_hbm,
                      src_v, dst_v, rows0, rows1, zrows, acc_sh, sem_i, sem0,
                      sem1):
    c = lax.axis_index("c")
    sid = lax.axis_index("s")
    wid = c * NS + sid

    if True:
        cp0 = pltpu.async_copy(src_hbm.at[wid], src_v, sem_i)
        cp1 = pltpu.async_copy(dst_hbm.at[wid], dst_v, sem_i)
        zrow_n = zrows.shape[0]

        def zbody(r, carry):
            for jj in range(8):
                zrows[r, pl.ds(jj * 16, 16)] = jnp.zeros((16,), jnp.float32)
            return carry

        lax.fori_loop(0, zrow_n, zbody, 0)
        zchunk = N_pad // NS

        def zcopy(r, carry):
            pltpu.sync_copy(
                zrows, acc_sh.at[pl.ds(sid * zchunk + r * zrow_n, zrow_n)])
            return carry

        lax.fori_loop(0, zchunk // zrow_n, zcopy, 0, unroll=8)
        cp0.wait()
        cp1.wait()
        plsc.subcore_barrier()

        def gather(j, buf, sem):
            return pltpu.make_async_copy(
                hs_hbm.at[src_v.at[pl.ds(j * K, K)]], buf, sem)

        def scat(buf, j):
            pltpu.sync_copy(buf, acc_sh.at[dst_v.at[j]], add=True)

        gather(0, rows0, sem0).start()

        def body(i, carry):
            j = 2 * i
            gather(j, rows0, sem0).wait()
            gather(j + 1, rows1, sem1).start()
            scat(rows0, j)
            gather(j + 1, rows1, sem1).wait()
            gather(j + 2, rows0, sem0).start()
            scat(rows1, j + 1)
            return carry

        lax.fori_loop(0, (n_chunks - 1) // 2, body, 0)
        gather(n_chunks - 1, rows0, sem0).wait()
        scat(rows0, n_chunks - 1)
        plsc.subcore_barrier()
        orows = N_pad // NS
        pltpu.sync_copy(acc_sh.at[pl.ds(sid * orows, orows)],
                        out_hbm.at[c, pl.ds(sid * orows, orows)])


def _mm_body(deg_ref, x_ref, w_ref, hs_ref, dinv_ref):
    deg = deg_ref[0] + deg_ref[1] + 1.0
    dinv = lax.rsqrt(deg)
    h = jnp.dot(x_ref[...], w_ref[...], preferred_element_type=jnp.float32)
    hs_ref[...] = h * dinv
    dinv_ref[...] = dinv


def _final_body(acc_ref, hs_ref, dinv_ref, b_ref, x_ref, out_ref):
    s = (acc_ref[0] + acc_ref[1] + hs_ref[...]) * dinv_ref[...] + b_ref[...]
    out_ref[...] = jnp.maximum(s, 0.0) + x_ref[...]


def kernel(x, edge_index, W, b):
    N, D = x.shape
    E = edge_index.shape[1]
    assert D == 128 and E % (NW * K) == 0
    n_chunks = E // (NW * K)
    N_pad = ((N + 16 * 640 - 1) // (16 * 640)) * (16 * 640)

    src = edge_index[0].reshape(NW, n_chunks * K)
    dst = edge_index[1].reshape(NW, n_chunks, K)

    mesh = plsc.VectorSubcoreMesh(core_axis_name="c", subcore_axis_name="s")

    deg_p = pl.kernel(
        functools.partial(_deg_kernel_body, N_pad, n_chunks),
        out_type=jax.ShapeDtypeStruct((NC, N_pad), jnp.float32),
        mesh=mesh,
        scratch_types=[
            pltpu.VMEM((n_chunks, K), jnp.int32),
            pltpu.VMEM((K,), jnp.float32),
            pltpu.VMEM((N_pad // NS,), jnp.float32),
            pltpu.VMEM_SHARED((N_pad,), jnp.float32),
            pltpu.SemaphoreType.DMA,
        ],
    )(dst)

    R = 400
    deg3 = deg_p.reshape(NC, N_pad, 1)
    hs, dinv = pl.pallas_call(
        _mm_body,
        grid=(N // R,),
        in_specs=[
            pl.BlockSpec((NC, R, 1), lambda i: (0, i, 0)),
            pl.BlockSpec((R, D), lambda i: (i, 0)),
            pl.BlockSpec((D, D), lambda i: (0, 0)),
        ],
        out_specs=[
            pl.BlockSpec((R, D), lambda i: (i, 0)),
            pl.BlockSpec((R, 1), lambda i: (i, 0)),
        ],
        out_shape=[
            jax.ShapeDtypeStruct((N, D), jnp.float32),
            jax.ShapeDtypeStruct((N, 1), jnp.float32),
        ],
    )(deg3, x, W)

    acc_p = pl.kernel(
        functools.partial(_edge_kernel_body, N, N_pad, n_chunks),
        out_type=jax.ShapeDtypeStruct((NC, N_pad, D), jnp.float32),
        mesh=mesh,
        scratch_types=[
            pltpu.VMEM((n_chunks * K,), jnp.int32),
            pltpu.VMEM((n_chunks, K), jnp.int32),
            pltpu.VMEM((K, D), jnp.float32),
            pltpu.VMEM((K, D), jnp.float32),
            pltpu.VMEM((8, D), jnp.float32),
            pltpu.VMEM_SHARED((N_pad, D), jnp.float32),
            pltpu.SemaphoreType.DMA,
            pltpu.SemaphoreType.DMA,
            pltpu.SemaphoreType.DMA,
        ],
    )(hs, src, dst)

    out = pl.pallas_call(
        _final_body,
        grid=(N // R,),
        in_specs=[
            pl.BlockSpec((NC, R, D), lambda i: (0, i, 0)),
            pl.BlockSpec((R, D), lambda i: (i, 0)),
            pl.BlockSpec((R, 1), lambda i: (i, 0)),
            pl.BlockSpec((1, D), lambda i: (0, 0)),
            pl.BlockSpec((R, D), lambda i: (i, 0)),
        ],
        out_specs=pl.BlockSpec((R, D), lambda i: (i, 0)),
        out_shape=jax.ShapeDtypeStruct((N, D), jnp.float32),
    )(acc_p, hs, dinv, b.reshape(1, D), x)
    return out


# speedup vs baseline: 29.8082x; 1.0470x over previous
import functools

import jax
import jax.numpy as jnp
from jax import lax
from jax.experimental import pallas as pl
from jax.experimental.pallas import tpu as pltpu
from jax.experimental.pallas import tpu_sc as plsc

NC, NS = 2, 16
NW = NC * NS
K = 80


def _deg_kernel_body(N_pad, n_chunks, dst_hbm, out_hbm, dst_v, ones_v, zero_v,
                     deg_sh, sem, sem_b):
    c = lax.axis_index("c")
    sid = lax.axis_index("s")
    wid = c * NS + sid

    if True:
        pltpu.async_copy(dst_hbm.at[wid], dst_v, sem).wait()
        zchunk = N_pad // NS

        def zbody(i, carry):
            zero_v[pl.ds(i * 16, 16)] = jnp.zeros((16,), jnp.float32)
            return carry

        lax.fori_loop(0, zchunk // 16, zbody, 0, unroll=8)
        pltpu.sync_copy(zero_v, deg_sh.at[pl.ds(sid * zchunk, zchunk)])
        for i in range(K // 16):
            ones_v[pl.ds(i * 16, 16)] = jnp.ones((16,), jnp.float32)
        plsc.subcore_barrier()

        def s_start(j, sem):
            pltpu.async_copy(ones_v, deg_sh.at[dst_v.at[j]], sem, add=True)

        def s_wait(sem):
            pltpu.make_async_copy(ones_v, deg_sh.at[dst_v.at[0]], sem).wait()

        s_start(0, sem)

        def body(i, carry):
            j = 2 * i + 1
            s_start(j, sem_b)
            s_wait(sem)
            s_start(j + 1, sem)
            s_wait(sem_b)
            return carry

        lax.fori_loop(0, (n_chunks - 1) // 2, body, 0)
        s_wait(sem)
        plsc.subcore_barrier()
        pltpu.sync_copy(deg_sh.at[pl.ds(sid * zchunk, zchunk)],
                        out_hbm.at[c, pl.ds(sid * zchunk, zchunk)])


def _edge_kernel_body(N, N_pad, n_chunks, hs_hbm, src_hbm, dst_hbm, out_hbm,
                      src_v, dst_v, rows0, rows1, acc_sh, sem_i, sem_g0,
                      sem_g1, sem_s0, sem_s1):
    c = lax.axis_index("c")
    sid = lax.axis_index("s")
    wid = c * NS + sid

    if True:
        cp0 = pltpu.async_copy(src_hbm.at[wid], src_v, sem_i)
        cp1 = pltpu.async_copy(dst_hbm.at[wid], dst_v, sem_i)

        def zbody(r, carry):
            for jj in range(8):
                z = jnp.zeros((16,), jnp.float32)
                rows0[r, pl.ds(jj * 16, 16)] = z
                rows1[r, pl.ds(jj * 16, 16)] = z
            return carry

        lax.fori_loop(0, K, zbody, 0)
        zchunk = N_pad // NS
        nz = zchunk // K

        def zslice(r):
            return acc_sh.at[pl.ds(sid * zchunk + r * K, K)]

        for r in range(nz):
            buf, sem = (rows0, sem_s0) if r % 2 == 0 else (rows1, sem_s1)
            pltpu.async_copy(buf, zslice(r), sem)
        for r in range(nz):
            buf, sem = (rows0, sem_s0) if r % 2 == 0 else (rows1, sem_s1)
            pltpu.make_async_copy(buf, zslice(r), sem).wait()
        cp0.wait()
        cp1.wait()
        plsc.subcore_barrier()

        def g_start(j, buf, sem):
            pltpu.async_copy(hs_hbm.at[src_v.at[pl.ds(j * K, K)]], buf, sem)

        def g_wait(j, buf, sem):
            pltpu.make_async_copy(hs_hbm.at[src_v.at[pl.ds(j * K, K)]], buf,
                                  sem).wait()

        def s_start(j, buf, sem):
            pltpu.async_copy(buf, acc_sh.at[dst_v.at[j]], sem, add=True)

        def s_wait(j, buf, sem):
            pltpu.make_async_copy(buf, acc_sh.at[dst_v.at[j]], sem).wait()

        g_start(0, rows0, sem_g0)
        g_wait(0, rows0, sem_g0)
        s_start(0, rows0, sem_s0)
        g_start(1, rows1, sem_g1)
        g_wait(1, rows1, sem_g1)
        s_start(1, rows1, sem_s1)
        s_wait(0, rows0, sem_s0)
        g_start(2, rows0, sem_g0)

        def body(i, carry):
            j = 2 * i
            s_wait(j - 1, rows1, sem_s1)
            g_start(j + 1, rows1, sem_g1)
            g_wait(j, rows0, sem_g0)
            s_start(j, rows0, sem_s0)
            g_wait(j + 1, rows1, sem_g1)
            s_start(j + 1, rows1, sem_s1)
            s_wait(j, rows0, sem_s0)
            g_start(j + 2, rows0, sem_g0)
            return carry

        lax.fori_loop(1, (n_chunks - 1) // 2, body, 0)
        j = n_chunks - 1
        g_wait(j, rows0, sem_g0)
        s_start(j, rows0, sem_s0)
        s_wait(j - 1, rows1, sem_s1)
        s_wait(j, rows0, sem_s0)
        plsc.subcore_barrier()
        orows = N_pad // NS
        pltpu.sync_copy(acc_sh.at[pl.ds(sid * orows, orows)],
                        out_hbm.at[c, pl.ds(sid * orows, orows)])


def _mm_body(deg_ref, x_ref, w_ref, hs_ref, dinv_ref):
    deg = deg_ref[0] + deg_ref[1] + 1.0
    dinv = lax.rsqrt(deg)
    h = jnp.dot(x_ref[...], w_ref[...], preferred_element_type=jnp.float32)
    hs_ref[...] = h * dinv
    dinv_ref[...] = dinv


def _final_body(acc_ref, hs_ref, dinv_ref, b_ref, x_ref, out_ref):
    s = (acc_ref[0] + acc_ref[1] + hs_ref[...]) * dinv_ref[...] + b_ref[...]
    out_ref[...] = jnp.maximum(s, 0.0) + x_ref[...]


def kernel(x, edge_index, W, b):
    N, D = x.shape
    E = edge_index.shape[1]
    assert D == 128 and E % (NW * K) == 0
    n_chunks = E // (NW * K)
    N_pad = ((N + 16 * 640 - 1) // (16 * 640)) * (16 * 640)

    src = edge_index[0].reshape(NW, n_chunks * K)
    dst = edge_index[1].reshape(NW, n_chunks, K)

    mesh = plsc.VectorSubcoreMesh(core_axis_name="c", subcore_axis_name="s")

    deg_p = pl.kernel(
        functools.partial(_deg_kernel_body, N_pad, n_chunks),
        out_type=jax.ShapeDtypeStruct((NC, N_pad), jnp.float32),
        mesh=mesh,
        scratch_types=[
            pltpu.VMEM((n_chunks, K), jnp.int32),
            pltpu.VMEM((K,), jnp.float32),
            pltpu.VMEM((N_pad // NS,), jnp.float32),
            pltpu.VMEM_SHARED((N_pad,), jnp.float32),
            pltpu.SemaphoreType.DMA,
            pltpu.SemaphoreType.DMA,
        ],
    )(dst)

    R = 400
    deg3 = deg_p.reshape(NC, N_pad, 1)
    hs, dinv = pl.pallas_call(
        _mm_body,
        grid=(N // R,),
        in_specs=[
            pl.BlockSpec((NC, R, 1), lambda i: (0, i, 0)),
            pl.BlockSpec((R, D), lambda i: (i, 0)),
            pl.BlockSpec((D, D), lambda i: (0, 0)),
        ],
        out_specs=[
            pl.BlockSpec((R, D), lambda i: (i, 0)),
            pl.BlockSpec((R, 1), lambda i: (i, 0)),
        ],
        out_shape=[
            jax.ShapeDtypeStruct((N, D), jnp.float32),
            jax.ShapeDtypeStruct((N, 1), jnp.float32),
        ],
    )(deg3, x, W)

    acc_p = pl.kernel(
        functools.partial(_edge_kernel_body, N, N_pad, n_chunks),
        out_type=jax.ShapeDtypeStruct((NC, N_pad, D), jnp.float32),
        mesh=mesh,
        scratch_types=[
            pltpu.VMEM((n_chunks * K,), jnp.int32),
            pltpu.VMEM((n_chunks, K), jnp.int32),
            pltpu.VMEM((K, D), jnp.float32),
            pltpu.VMEM((K, D), jnp.float32),
            pltpu.VMEM_SHARED((N_pad, D), jnp.float32),
            pltpu.SemaphoreType.DMA,
            pltpu.SemaphoreType.DMA,
            pltpu.SemaphoreType.DMA,
            pltpu.SemaphoreType.DMA,
            pltpu.SemaphoreType.DMA,
        ],
    )(hs, src, dst)

    out = pl.pallas_call(
        _final_body,
        grid=(N // R,),
        in_specs=[
            pl.BlockSpec((NC, R, D), lambda i: (0, i, 0)),
            pl.BlockSpec((R, D), lambda i: (i, 0)),
            pl.BlockSpec((R, 1), lambda i: (i, 0)),
            pl.BlockSpec((1, D), lambda i: (0, 0)),
            pl.BlockSpec((R, D), lambda i: (i, 0)),
        ],
        out_specs=pl.BlockSpec((R, D), lambda i: (i, 0)),
        out_shape=jax.ShapeDtypeStruct((N, D), jnp.float32),
    )(acc_p, hs, dinv, b.reshape(1, D), x)
    return out


# speedup vs baseline: 31.1246x; 1.0442x over previous
import functools

import jax
import jax.numpy as jnp
from jax import lax
from jax.experimental import pallas as pl
from jax.experimental.pallas import tpu as pltpu
from jax.experimental.pallas import tpu_sc as plsc

NC, NS = 2, 16
NW = NC * NS
K = 80


def _deg_kernel_body(N_pad, n_chunks, ei_hbm, out_hbm, dst_v, ones_v, zero_v,
                     deg_sh, sem, sem_b):
    c = lax.axis_index("c")
    sid = lax.axis_index("s")
    wid = c * NS + sid

    if True:
        pltpu.async_copy(ei_hbm.at[NW + wid], dst_v, sem).wait()
        zchunk = N_pad // NS

        def zbody(i, carry):
            zero_v[pl.ds(i * 16, 16)] = jnp.zeros((16,), jnp.float32)
            return carry

        lax.fori_loop(0, zchunk // 16, zbody, 0, unroll=8)
        pltpu.sync_copy(zero_v, deg_sh.at[pl.ds(sid * zchunk, zchunk)])
        for i in range(K // 16):
            ones_v[pl.ds(i * 16, 16)] = jnp.ones((16,), jnp.float32)
        plsc.subcore_barrier()

        def s_start(j, sem):
            pltpu.async_copy(ones_v, deg_sh.at[dst_v.at[j]], sem, add=True)

        def s_wait(sem):
            pltpu.make_async_copy(ones_v, deg_sh.at[dst_v.at[0]], sem).wait()

        s_start(0, sem)

        def body(i, carry):
            j = 2 * i + 1
            s_start(j, sem_b)
            s_wait(sem)
            s_start(j + 1, sem)
            s_wait(sem_b)
            return carry

        lax.fori_loop(0, (n_chunks - 1) // 2, body, 0)
        s_wait(sem)
        plsc.subcore_barrier()
        pltpu.sync_copy(deg_sh.at[pl.ds(sid * zchunk, zchunk)],
                        out_hbm.at[c, pl.ds(sid * zchunk, zchunk)])


def _edge_kernel_body(N, N_pad, n_chunks, hs_hbm, ei2_hbm, ei3_hbm, out_hbm,
                      src_v, dst_v, rows0, rows1, acc_sh, sem_i, sem_g0,
                      sem_g1, sem_s0, sem_s1):
    c = lax.axis_index("c")
    sid = lax.axis_index("s")
    wid = c * NS + sid

    if True:
        cp0 = pltpu.async_copy(ei2_hbm.at[wid], src_v, sem_i)
        cp1 = pltpu.async_copy(ei3_hbm.at[NW + wid], dst_v, sem_i)

        def zbody(r, carry):
            for jj in range(8):
                z = jnp.zeros((16,), jnp.float32)
                rows0[r, pl.ds(jj * 16, 16)] = z
                rows1[r, pl.ds(jj * 16, 16)] = z
            return carry

        lax.fori_loop(0, K, zbody, 0)
        zchunk = N_pad // NS
        nz = zchunk // K

        def zslice(r):
            return acc_sh.at[pl.ds(sid * zchunk + r * K, K)]

        for r in range(nz):
            buf, sem = (rows0, sem_s0) if r % 2 == 0 else (rows1, sem_s1)
            pltpu.async_copy(buf, zslice(r), sem)
        for r in range(nz):
            buf, sem = (rows0, sem_s0) if r % 2 == 0 else (rows1, sem_s1)
            pltpu.make_async_copy(buf, zslice(r), sem).wait()
        cp0.wait()
        cp1.wait()
        plsc.subcore_barrier()

        def g_start(j, buf, sem):
            pltpu.async_copy(hs_hbm.at[src_v.at[pl.ds(j * K, K)]], buf, sem)

        def g_wait(j, buf, sem):
            pltpu.make_async_copy(hs_hbm.at[src_v.at[pl.ds(j * K, K)]], buf,
                                  sem).wait()

        def s_start(j, buf, sem):
            pltpu.async_copy(buf, acc_sh.at[dst_v.at[j]], sem, add=True)

        def s_wait(j, buf, sem):
            pltpu.make_async_copy(buf, acc_sh.at[dst_v.at[j]], sem).wait()

        g_start(0, rows0, sem_g0)
        g_wait(0, rows0, sem_g0)
        s_start(0, rows0, sem_s0)
        g_start(1, rows1, sem_g1)
        g_wait(1, rows1, sem_g1)
        s_start(1, rows1, sem_s1)
        s_wait(0, rows0, sem_s0)
        g_start(2, rows0, sem_g0)

        def body(i, carry):
            j = 2 * i
            s_wait(j - 1, rows1, sem_s1)
            g_start(j + 1, rows1, sem_g1)
            g_wait(j, rows0, sem_g0)
            s_start(j, rows0, sem_s0)
            g_wait(j + 1, rows1, sem_g1)
            s_start(j + 1, rows1, sem_s1)
            s_wait(j, rows0, sem_s0)
            g_start(j + 2, rows0, sem_g0)
            return carry

        lax.fori_loop(1, (n_chunks - 1) // 2, body, 0)
        j = n_chunks - 1
        g_wait(j, rows0, sem_g0)
        s_start(j, rows0, sem_s0)
        s_wait(j - 1, rows1, sem_s1)
        s_wait(j, rows0, sem_s0)
        plsc.subcore_barrier()
        orows = N_pad // NS
        pltpu.sync_copy(acc_sh.at[pl.ds(sid * orows, orows)],
                        out_hbm.at[c, pl.ds(sid * orows, orows)])


def _mm_body(deg_ref, x_ref, w_ref, hs_ref, dinv_ref):
    deg = deg_ref[0] + deg_ref[1] + 1.0
    dinv = lax.rsqrt(deg)
    h = jnp.dot(x_ref[...], w_ref[...], preferred_element_type=jnp.float32)
    hs_ref[...] = h * dinv
    dinv_ref[...] = dinv


def _final_body(acc_ref, hs_ref, dinv_ref, b_ref, x_ref, out_ref):
    s = (acc_ref[0] + acc_ref[1] + hs_ref[...]) * dinv_ref[...] + b_ref[...]
    out_ref[...] = jnp.maximum(s, 0.0) + x_ref[...]


def kernel(x, edge_index, W, b):
    N, D = x.shape
    E = edge_index.shape[1]
    assert D == 128 and E % (NW * K) == 0
    n_chunks = E // (NW * K)
    N_pad = ((N + 16 * 640 - 1) // (16 * 640)) * (16 * 640)

    ei2 = edge_index.reshape(2 * NW, n_chunks * K)
    ei3 = edge_index.reshape(2 * NW, n_chunks, K)

    mesh = plsc.VectorSubcoreMesh(core_axis_name="c", subcore_axis_name="s")

    deg_p = pl.kernel(
        functools.partial(_deg_kernel_body, N_pad, n_chunks),
        out_type=jax.ShapeDtypeStruct((NC, N_pad), jnp.float32),
        mesh=mesh,
        scratch_types=[
            pltpu.VMEM((n_chunks, K), jnp.int32),
            pltpu.VMEM((K,), jnp.float32),
            pltpu.VMEM((N_pad // NS,), jnp.float32),
            pltpu.VMEM_SHARED((N_pad,), jnp.float32),
            pltpu.SemaphoreType.DMA,
            pltpu.SemaphoreType.DMA,
        ],
    )(ei3)

    R = 400
    deg3 = deg_p.reshape(NC, N_pad, 1)
    hs, dinv = pl.pallas_call(
        _mm_body,
        grid=(N // R,),
        in_specs=[
            pl.BlockSpec((NC, R, 1), lambda i: (0, i, 0)),
            pl.BlockSpec((R, D), lambda i: (i, 0)),
            pl.BlockSpec((D, D), lambda i: (0, 0)),
        ],
        out_specs=[
            pl.BlockSpec((R, D), lambda i: (i, 0)),
            pl.BlockSpec((R, 1), lambda i: (i, 0)),
        ],
        out_shape=[
            jax.ShapeDtypeStruct((N, D), jnp.float32),
            jax.ShapeDtypeStruct((N, 1), jnp.float32),
        ],
    )(deg3, x, W)

    acc_p = pl.kernel(
        functools.partial(_edge_kernel_body, N, N_pad, n_chunks),
        out_type=jax.ShapeDtypeStruct((NC, N_pad, D), jnp.float32),
        mesh=mesh,
        scratch_types=[
            pltpu.VMEM((n_chunks * K,), jnp.int32),
            pltpu.VMEM((n_chunks, K), jnp.int32),
            pltpu.VMEM((K, D), jnp.float32),
            pltpu.VMEM((K, D), jnp.float32),
            pltpu.VMEM_SHARED((N_pad, D), jnp.float32),
            pltpu.SemaphoreType.DMA,
            pltpu.SemaphoreType.DMA,
            pltpu.SemaphoreType.DMA,
            pltpu.SemaphoreType.DMA,
            pltpu.SemaphoreType.DMA,
        ],
    )(hs, ei2, ei3)

    out = pl.pallas_call(
        _final_body,
        grid=(N // R,),
        in_specs=[
            pl.BlockSpec((NC, R, D), lambda i: (0, i, 0)),
            pl.BlockSpec((R, D), lambda i: (i, 0)),
            pl.BlockSpec((R, 1), lambda i: (i, 0)),
            pl.BlockSpec((1, D), lambda i: (0, 0)),
            pl.BlockSpec((R, D), lambda i: (i, 0)),
        ],
        out_specs=pl.BlockSpec((R, D), lambda i: (i, 0)),
        out_shape=jax.ShapeDtypeStruct((N, D), jnp.float32),
    )(acc_p, hs, dinv, b.reshape(1, D), x)
    return out


# speedup vs baseline: 35.3969x; 1.1373x over previous
import functools

import jax
import jax.numpy as jnp
from jax import lax
from jax.experimental import pallas as pl
from jax.experimental.pallas import tpu as pltpu
from jax.experimental.pallas import tpu_sc as plsc

NC, NS = 2, 16
NW = NC * NS
K = 80


def _deg_kernel_body(N_pad, n_chunks, ei_hbm, out_hbm, dst_v, ones_v, zero_v,
                     deg_sh, sem, sem_b):
    c = lax.axis_index("c")
    sid = lax.axis_index("s")
    wid = c * NS + sid

    if True:
        pltpu.async_copy(ei_hbm.at[NW + wid], dst_v, sem).wait()
        zchunk = N_pad // NS

        def zbody(i, carry):
            zero_v[pl.ds(i * 16, 16)] = jnp.zeros((16,), jnp.float32)
            return carry

        lax.fori_loop(0, zchunk // 16, zbody, 0, unroll=8)
        pltpu.sync_copy(zero_v, deg_sh.at[pl.ds(sid * zchunk, zchunk)])
        for i in range(K // 16):
            ones_v[pl.ds(i * 16, 16)] = jnp.ones((16,), jnp.float32)
        plsc.subcore_barrier()

        def s_start(j, sem):
            pltpu.async_copy(ones_v, deg_sh.at[dst_v.at[j]], sem, add=True)

        def s_wait(sem):
            pltpu.make_async_copy(ones_v, deg_sh.at[dst_v.at[0]], sem).wait()

        s_start(0, sem)

        def body(i, carry):
            j = 2 * i + 1
            s_start(j, sem_b)
            s_wait(sem)
            s_start(j + 1, sem)
            s_wait(sem_b)
            return carry

        lax.fori_loop(0, (n_chunks - 1) // 2, body, 0)
        s_wait(sem)
        plsc.subcore_barrier()
        pltpu.sync_copy(deg_sh.at[pl.ds(sid * zchunk, zchunk)],
                        out_hbm.at[c, pl.ds(sid * zchunk, zchunk)])


def _edge_kernel_body(N, N_pad, n_chunks, hs_hbm, ei2_hbm, ei3_hbm, out_hbm,
                      src_v, dst_v, rows0, rows1, acc_sh, sem_i, sem_g0,
                      sem_g1, sem_s0, sem_s1):
    c = lax.axis_index("c")
    sid = lax.axis_index("s")
    wid = c * NS + sid

    if True:
        cp0 = pltpu.async_copy(ei2_hbm.at[wid], src_v, sem_i)
        cp1 = pltpu.async_copy(ei3_hbm.at[NW + wid], dst_v, sem_i)

        def zbody(r, carry):
            for jj in range(8):
                z = jnp.zeros((16,), jnp.float32)
                rows0[r, pl.ds(jj * 16, 16)] = z
                rows1[r, pl.ds(jj * 16, 16)] = z
            return carry

        lax.fori_loop(0, K, zbody, 0)
        zchunk = N_pad // NS
        nz = zchunk // K

        def zslice(r):
            return acc_sh.at[pl.ds(sid * zchunk + r * K, K)]

        for r in range(nz):
            buf, sem = (rows0, sem_s0) if r % 2 == 0 else (rows1, sem_s1)
            pltpu.async_copy(buf, zslice(r), sem)
        for r in range(nz):
            buf, sem = (rows0, sem_s0) if r % 2 == 0 else (rows1, sem_s1)
            pltpu.make_async_copy(buf, zslice(r), sem).wait()
        cp0.wait()
        cp1.wait()
        plsc.subcore_barrier()

        def g_start(j, buf, sem):
            pltpu.async_copy(hs_hbm.at[src_v.at[pl.ds(j * K, K)]], buf, sem)

        def g_wait(j, buf, sem):
            pltpu.make_async_copy(hs_hbm.at[src_v.at[pl.ds(j * K, K)]], buf,
                                  sem).wait()

        def s_start(j, buf, sem):
            pltpu.async_copy(buf, acc_sh.at[dst_v.at[j]], sem, add=True)

        def s_wait(j, buf, sem):
            pltpu.make_async_copy(buf, acc_sh.at[dst_v.at[j]], sem).wait()

        g_start(0, rows0, sem_g0)
        g_wait(0, rows0, sem_g0)
        s_start(0, rows0, sem_s0)
        g_start(1, rows1, sem_g1)
        g_wait(1, rows1, sem_g1)
        s_start(1, rows1, sem_s1)
        s_wait(0, rows0, sem_s0)
        g_start(2, rows0, sem_g0)

        def body(i, carry):
            j = 2 * i
            s_wait(j - 1, rows1, sem_s1)
            g_start(j + 1, rows1, sem_g1)
            g_wait(j, rows0, sem_g0)
            s_start(j, rows0, sem_s0)
            g_wait(j + 1, rows1, sem_g1)
            s_start(j + 1, rows1, sem_s1)
            s_wait(j, rows0, sem_s0)
            g_start(j + 2, rows0, sem_g0)
            return carry

        lax.fori_loop(1, (n_chunks - 1) // 2, body, 0)
        j = n_chunks - 1
        g_wait(j, rows0, sem_g0)
        s_start(j, rows0, sem_s0)
        s_wait(j - 1, rows1, sem_s1)
        s_wait(j, rows0, sem_s0)
        plsc.subcore_barrier()
        orows = N_pad // NS
        pltpu.sync_copy(acc_sh.at[pl.ds(sid * orows, orows)],
                        out_hbm.at[c, pl.ds(sid * orows, orows)])


def _mm_body(deg_ref, x_ref, w_ref, hs_ref, dinv_ref):
    deg = deg_ref[0] + deg_ref[1] + 1.0
    dinv = lax.rsqrt(deg)
    dinv_ref[...] = dinv[None, :]
    h = jnp.dot(x_ref[...], w_ref[...], preferred_element_type=jnp.float32)
    hs_ref[...] = h * dinv[:, None]


def _final_body(acc_ref, hs_ref, dinv_ref, b_ref, x_ref, out_ref):
    dinv = dinv_ref[0][:, None]
    s = (acc_ref[0] + acc_ref[1] + hs_ref[...]) * dinv + b_ref[...]
    out_ref[...] = jnp.maximum(s, 0.0) + x_ref[...]


def kernel(x, edge_index, W, b):
    N, D = x.shape
    E = edge_index.shape[1]
    assert D == 128 and E % (NW * K) == 0
    n_chunks = E // (NW * K)
    N_pad = ((N + 16 * 640 - 1) // (16 * 640)) * (16 * 640)

    ei2 = edge_index.reshape(2 * NW, n_chunks * K)
    ei3 = edge_index.reshape(2 * NW, n_chunks, K)

    mesh = plsc.VectorSubcoreMesh(core_axis_name="c", subcore_axis_name="s")

    deg_p = pl.kernel(
        functools.partial(_deg_kernel_body, N_pad, n_chunks),
        out_type=jax.ShapeDtypeStruct((NC, N_pad), jnp.float32),
        mesh=mesh,
        scratch_types=[
            pltpu.VMEM((n_chunks, K), jnp.int32),
            pltpu.VMEM((K,), jnp.float32),
            pltpu.VMEM((N_pad // NS,), jnp.float32),
            pltpu.VMEM_SHARED((N_pad,), jnp.float32),
            pltpu.SemaphoreType.DMA,
            pltpu.SemaphoreType.DMA,
        ],
    )(ei3)

    R = 1280
    hs, dinv = pl.pallas_call(
        _mm_body,
        grid=(N_pad // R,),
        in_specs=[
            pl.BlockSpec((NC, R), lambda i: (0, i)),
            pl.BlockSpec((R, D), lambda i: (i, 0)),
            pl.BlockSpec((D, D), lambda i: (0, 0)),
        ],
        out_specs=[
            pl.BlockSpec((R, D), lambda i: (i, 0)),
            pl.BlockSpec((1, R), lambda i: (0, i)),
        ],
        out_shape=[
            jax.ShapeDtypeStruct((N, D), jnp.float32),
            jax.ShapeDtypeStruct((1, N), jnp.float32),
        ],
    )(deg_p, x, W)

    acc_p = pl.kernel(
        functools.partial(_edge_kernel_body, N, N_pad, n_chunks),
        out_type=jax.ShapeDtypeStruct((NC, N_pad, D), jnp.float32),
        mesh=mesh,
        scratch_types=[
            pltpu.VMEM((n_chunks * K,), jnp.int32),
            pltpu.VMEM((n_chunks, K), jnp.int32),
            pltpu.VMEM((K, D), jnp.float32),
            pltpu.VMEM((K, D), jnp.float32),
            pltpu.VMEM_SHARED((N_pad, D), jnp.float32),
            pltpu.SemaphoreType.DMA,
            pltpu.SemaphoreType.DMA,
            pltpu.SemaphoreType.DMA,
            pltpu.SemaphoreType.DMA,
            pltpu.SemaphoreType.DMA,
        ],
    )(hs, ei2, ei3)

    out = pl.pallas_call(
        _final_body,
        grid=(N_pad // R,),
        in_specs=[
            pl.BlockSpec((NC, R, D), lambda i: (0, i, 0)),
            pl.BlockSpec((R, D), lambda i: (i, 0)),
            pl.BlockSpec((1, R), lambda i: (0, i)),
            pl.BlockSpec((1, D), lambda i: (0, 0)),
            pl.BlockSpec((R, D), lambda i: (i, 0)),
        ],
        out_specs=pl.BlockSpec((R, D), lambda i: (i, 0)),
        out_shape=jax.ShapeDtypeStruct((N, D), jnp.float32),
    )(acc_p, hs, dinv, b.reshape(1, D), x)
    return out
